# Initial kernel scaffold; baseline (speedup 1.0000x reference)
import jax
import jax.numpy as jnp
from jax.experimental import pallas as pl


def kernel(input, action_embedding):
    raise NotImplementedError("write your pallas kernel here")


# speedup vs baseline: 2.9720x; 2.9720x over previous
import functools

import jax
import jax.numpy as jnp
from jax import lax
from jax.experimental import pallas as pl
from jax.experimental.pallas import tpu as pltpu
from jax.experimental.pallas import tpu_sc as plsc

NUM_ACTIONS = 100000
LATENT_DIM = 128
BATCH = 4096
HIST = 50

_INFO = plsc.get_sparse_core_info()
_NC, _NS = _INFO.num_cores, _INFO.num_subcores
_NW = _NC * _NS

_TOTAL = BATCH * HIST
_CHUNK = 128
_NROWS = _TOTAL // _CHUNK
_ROWS_PER_W = _NROWS // _NW


def _gather_body(idx_hbm, table_hbm, out_hbm, idx_v, rows_v, sem):
    wid = lax.axis_index("s") * _NC + lax.axis_index("c")
    base = wid * _ROWS_PER_W
    pltpu.sync_copy(idx_hbm.at[wid], idx_v)

    def step(j, carry):
        pltpu.async_copy(table_hbm.at[idx_v.at[j]], rows_v, sem).wait()
        pltpu.sync_copy(rows_v, out_hbm.at[pl.ds((base + j) * _CHUNK, _CHUNK), :])
        return carry

    lax.fori_loop(0, _ROWS_PER_W, step, 0)


@functools.partial(jax.jit, static_argnums=())
def _embed_gather(idx2d, table):
    mesh = plsc.VectorSubcoreMesh(core_axis_name="c", subcore_axis_name="s")
    f = functools.partial(
        pl.kernel,
        mesh=mesh,
        out_type=jax.ShapeDtypeStruct((_TOTAL, LATENT_DIM), jnp.float32),
        scratch_types=[
            pltpu.VMEM((_ROWS_PER_W, _CHUNK), jnp.int32),
            pltpu.VMEM((_CHUNK, LATENT_DIM), jnp.float32),
            pltpu.SemaphoreType.DMA,
        ],
    )(_gather_body)
    return f(idx2d, table)


def kernel(input, action_embedding):
    idx2d = input.astype(jnp.int32).reshape(_NW, _ROWS_PER_W, _CHUNK)
    out = _embed_gather(idx2d, action_embedding)
    return out.reshape(BATCH, HIST, LATENT_DIM)


# speedup vs baseline: 3.3064x; 1.1125x over previous
import functools

import jax
import jax.numpy as jnp
from jax import lax
from jax.experimental import pallas as pl
from jax.experimental.pallas import tpu as pltpu
from jax.experimental.pallas import tpu_sc as plsc

NUM_ACTIONS = 100000
LATENT_DIM = 128
BATCH = 4096
HIST = 50

_INFO = plsc.get_sparse_core_info()
_NC, _NS = _INFO.num_cores, _INFO.num_subcores
_NW = _NC * _NS

_TOTAL = BATCH * HIST
_CHUNK = 128
_NROWS = _TOTAL // _CHUNK
_ROWS_PER_W = _NROWS // _NW


_NBUF = 5
_NITER = _ROWS_PER_W // _NBUF


def _gather_body(idx_hbm, table_hbm, out_hbm, idx_v, rows_v, semg, semc):
    wid = lax.axis_index("s") * _NC + lax.axis_index("c")
    base = wid * _ROWS_PER_W
    pltpu.sync_copy(idx_hbm.at[wid], idx_v)

    for b in range(_NBUF):
        pltpu.async_copy(table_hbm.at[idx_v.at[b]], rows_v.at[b], semg.at[b])

    def step(t, carry):
        j0 = t * _NBUF
        for b in range(_NBUF):
            pltpu.make_async_copy(
                table_hbm.at[idx_v.at[j0 + b]], rows_v.at[b], semg.at[b]
            ).wait()
            pltpu.async_copy(
                rows_v.at[b],
                out_hbm.at[pl.ds((base + j0 + b) * _CHUNK, _CHUNK), :],
                semc.at[b],
            )

        @pl.when(t < _NITER - 1)
        def _refill():
            for b in range(_NBUF):
                j = j0 + _NBUF + b
                pltpu.make_async_copy(
                    rows_v.at[b],
                    out_hbm.at[pl.ds((base + j0 + b) * _CHUNK, _CHUNK), :],
                    semc.at[b],
                ).wait()
                pltpu.async_copy(table_hbm.at[idx_v.at[j]], rows_v.at[b], semg.at[b])

        return carry

    lax.fori_loop(0, _NITER, step, 0)

    j0 = (_NITER - 1) * _NBUF
    for b in range(_NBUF):
        pltpu.make_async_copy(
            rows_v.at[b],
            out_hbm.at[pl.ds((base + j0 + b) * _CHUNK, _CHUNK), :],
            semc.at[b],
        ).wait()


@functools.partial(jax.jit, static_argnums=())
def _embed_gather(idx2d, table):
    mesh = plsc.VectorSubcoreMesh(core_axis_name="c", subcore_axis_name="s")
    f = functools.partial(
        pl.kernel,
        mesh=mesh,
        out_type=jax.ShapeDtypeStruct((_TOTAL, LATENT_DIM), jnp.float32),
        scratch_types=[
            pltpu.VMEM((_ROWS_PER_W, _CHUNK), jnp.int32),
            pltpu.VMEM((_NBUF, _CHUNK, LATENT_DIM), jnp.float32),
            pltpu.SemaphoreType.DMA((_NBUF,)),
            pltpu.SemaphoreType.DMA((_NBUF,)),
        ],
    )(_gather_body)
    return f(idx2d, table)


def kernel(input, action_embedding):
    idx2d = input.astype(jnp.int32).reshape(_NW, _ROWS_PER_W, _CHUNK)
    out = _embed_gather(idx2d, action_embedding)
    return out.reshape(BATCH, HIST, LATENT_DIM)


# speedup vs baseline: 5.8658x; 1.7741x over previous
import functools

import jax
import jax.numpy as jnp
from jax import lax
from jax.experimental import pallas as pl
from jax.experimental.pallas import tpu as pltpu
from jax.experimental.pallas import tpu_sc as plsc

NUM_ACTIONS = 100000
LATENT_DIM = 128
BATCH = 4096
HIST = 50

_INFO = plsc.get_sparse_core_info()
_NC, _NS = _INFO.num_cores, _INFO.num_subcores
_NW = _NC * _NS
_B_PER_W = BATCH // _NW

_G = 4
_NBUF = 4
_NGROUP = _B_PER_W // _G
_NITER = _NGROUP // _NBUF


def _gather_body(idx_hbm, table_hbm, out_hbm, idx_v, rows_v, semg, semc):
    wid = lax.axis_index("s") * _NC + lax.axis_index("c")
    wbase = wid * _B_PER_W
    pltpu.sync_copy(idx_hbm.at[pl.ds(wbase, _B_PER_W), :], idx_v)

    def fire_group(g_abs, b):
        for i in range(_G):
            pltpu.async_copy(
                table_hbm.at[idx_v.at[g_abs * _G + i]], rows_v.at[b, i], semg.at[b]
            )

    def wait_group(g_abs, b):
        for i in range(_G):
            pltpu.make_async_copy(
                table_hbm.at[idx_v.at[g_abs * _G + i]], rows_v.at[b, i], semg.at[b]
            ).wait()

    def copyout(g_abs, b):
        return pltpu.make_async_copy(
            rows_v.at[b], out_hbm.at[pl.ds(wbase + g_abs * _G, _G)], semc.at[b]
        )

    for b in range(_NBUF):
        fire_group(b, b)

    def step(t, carry):
        g0 = t * _NBUF
        for b in range(_NBUF):
            wait_group(g0 + b, b)
            copyout(g0 + b, b).start()

        @pl.when(t < _NITER - 1)
        def _refill():
            for b in range(_NBUF):
                copyout(g0 + b, b).wait()
                fire_group(g0 + _NBUF + b, b)

        return carry

    lax.fori_loop(0, _NITER, step, 0)

    g0 = (_NITER - 1) * _NBUF
    for b in range(_NBUF):
        copyout(g0 + b, b).wait()


@jax.jit
def _embed_gather(idx, table):
    mesh = plsc.VectorSubcoreMesh(core_axis_name="c", subcore_axis_name="s")
    f = functools.partial(
        pl.kernel,
        mesh=mesh,
        out_type=jax.ShapeDtypeStruct((BATCH, HIST, LATENT_DIM), jnp.float32),
        scratch_types=[
            pltpu.VMEM((_B_PER_W, HIST), jnp.int32),
            pltpu.VMEM((_NBUF, _G, HIST, LATENT_DIM), jnp.float32),
            pltpu.SemaphoreType.DMA((_NBUF,)),
            pltpu.SemaphoreType.DMA((_NBUF,)),
        ],
    )(_gather_body)
    return f(idx, table)


def kernel(input, action_embedding):
    return _embed_gather(input.astype(jnp.int32), action_embedding)


# speedup vs baseline: 10.4318x; 1.7784x over previous
import functools

import jax
import jax.numpy as jnp
from jax import lax
from jax.experimental import pallas as pl
from jax.experimental.pallas import tpu as pltpu
from jax.experimental.pallas import tpu_sc as plsc

NUM_ACTIONS = 100000
LATENT_DIM = 128
BATCH = 4096
HIST = 50

_INFO = plsc.get_sparse_core_info()
_NC, _NS = _INFO.num_cores, _INFO.num_subcores
_NW = _NC * _NS
_BW = BATCH // _NW

_NBUF = 5
_NITER = HIST // _NBUF


def _gather_body(idx_hbm, table_hbm, out_hbm, idx_v, rows_v, semg, semc):
    wid = lax.axis_index("s") * _NC + lax.axis_index("c")
    wbase = wid * _BW
    pltpu.sync_copy(idx_hbm.at[:, pl.ds(wbase, _BW)], idx_v)

    def gather(h, b):
        return pltpu.make_async_copy(
            table_hbm.at[idx_v.at[h]], rows_v.at[b], semg.at[b]
        )

    def copyout(h, b):
        return pltpu.make_async_copy(
            rows_v.at[b], out_hbm.at[h, pl.ds(wbase, _BW)], semc.at[b]
        )

    for b in range(_NBUF):
        gather(b, b).start()

    def step(t, carry):
        h0 = t * _NBUF
        for b in range(_NBUF):
            gather(h0 + b, b).wait()
            copyout(h0 + b, b).start()

        @pl.when(t < _NITER - 1)
        def _refill():
            for b in range(_NBUF):
                copyout(h0 + b, b).wait()
                gather(h0 + _NBUF + b, b).start()

        return carry

    lax.fori_loop(0, _NITER, step, 0)

    h0 = (_NITER - 1) * _NBUF
    for b in range(_NBUF):
        copyout(h0 + b, b).wait()


@jax.jit
def _embed_gather(idx_t, table):
    mesh = plsc.VectorSubcoreMesh(core_axis_name="c", subcore_axis_name="s")
    f = functools.partial(
        pl.kernel,
        mesh=mesh,
        out_type=jax.ShapeDtypeStruct((HIST, BATCH, LATENT_DIM), jnp.float32),
        scratch_types=[
            pltpu.VMEM((HIST, _BW), jnp.int32),
            pltpu.VMEM((_NBUF, _BW, LATENT_DIM), jnp.float32),
            pltpu.SemaphoreType.DMA((_NBUF,)),
            pltpu.SemaphoreType.DMA((_NBUF,)),
        ],
    )(_gather_body)
    return f(idx_t, table)


def kernel(input, action_embedding):
    idx_t = input.astype(jnp.int32).T
    out_t = _embed_gather(idx_t, action_embedding)
    return out_t.transpose(1, 0, 2)


# speedup vs baseline: 10.7021x; 1.0259x over previous
import functools

import jax
import jax.numpy as jnp
from jax import lax
from jax.experimental import pallas as pl
from jax.experimental.pallas import tpu as pltpu
from jax.experimental.pallas import tpu_sc as plsc

NUM_ACTIONS = 100000
LATENT_DIM = 128
BATCH = 4096
HIST = 50

_INFO = plsc.get_sparse_core_info()
_NC, _NS = _INFO.num_cores, _INFO.num_subcores
_NW = _NC * _NS
_BW = BATCH // _NW

_CW = 64
_CPH = _BW // _CW
_NCHUNK = HIST * _CPH
_NBUF = 10
_NITER = _NCHUNK // _NBUF


def _gather_body(idx_hbm, table_hbm, out_hbm, idx_v, rows_v, semg, semc):
    wid = lax.axis_index("s") * _NC + lax.axis_index("c")
    wbase = wid * _BW
    pltpu.sync_copy(idx_hbm.at[:, pl.ds(wbase, _BW)], idx_v)

    def gather(j, b):
        h = j // _CPH
        col = (j % _CPH) * _CW
        return pltpu.make_async_copy(
            table_hbm.at[idx_v.at[h, pl.ds(col, _CW)]], rows_v.at[b], semg.at[b]
        )

    def copyout(j, b):
        h = j // _CPH
        col = (j % _CPH) * _CW
        return pltpu.make_async_copy(
            rows_v.at[b], out_hbm.at[h, pl.ds(wbase + col, _CW)], semc.at[b]
        )

    for b in range(_NBUF):
        gather(b, b).start()

    def step(t, carry):
        j0 = t * _NBUF
        for b in range(_NBUF):
            gather(j0 + b, b).wait()
            copyout(j0 + b, b).start()

        @pl.when(t < _NITER - 1)
        def _refill():
            for b in range(_NBUF):
                copyout(j0 + b, b).wait()
                gather(j0 + _NBUF + b, b).start()

        return carry

    lax.fori_loop(0, _NITER, step, 0)

    j0 = (_NITER - 1) * _NBUF
    for b in range(_NBUF):
        copyout(j0 + b, b).wait()


@jax.jit
def _embed_gather(idx_t, table):
    mesh = plsc.VectorSubcoreMesh(core_axis_name="c", subcore_axis_name="s")
    f = functools.partial(
        pl.kernel,
        mesh=mesh,
        out_type=jax.ShapeDtypeStruct((HIST, BATCH, LATENT_DIM), jnp.float32),
        scratch_types=[
            pltpu.VMEM((HIST, _BW), jnp.int32),
            pltpu.VMEM((_NBUF, _CW, LATENT_DIM), jnp.float32),
            pltpu.SemaphoreType.DMA((_NBUF,)),
            pltpu.SemaphoreType.DMA((_NBUF,)),
        ],
    )(_gather_body)
    return f(idx_t, table)


def kernel(input, action_embedding):
    idx_t = input.astype(jnp.int32).T
    out_t = _embed_gather(idx_t, action_embedding)
    return out_t.transpose(1, 0, 2)
